# baseline (speedup 1.0000x reference)
import functools

import jax
import jax.numpy as jnp
from jax import lax
from jax.experimental import pallas as pl
from jax.experimental.pallas import tpu as pltpu
from jax.experimental.pallas import tpu_sc as plsc

N_ANGLE = 6
N_SIGMA = 4
N_ANG_PARTS = 6
N_RAD_PARTS = 3
N_PARTS = N_ANG_PARTS * N_RAD_PARTS
PIX = 4 * 24
DESC = N_ANG_PARTS * N_RAD_PARTS * N_ANGLE

OUT_W = 128
GROUP = 16
N_WORKERS = 32


def _tc_body(x_ref, h_ref, v_ref):
    x = x_ref[...].reshape(N_ANGLE * N_SIGMA, PIX, x_ref.shape[-1])
    acc = None
    for s in range(N_SIGMA):
        m = x[s]
        sh = jnp.zeros(m.shape, jnp.int32)
        for a in range(1, N_ANGLE):
            xa = x[a * N_SIGMA + s]
            gt = xa > m
            m = jnp.maximum(m, xa)
            sh = jnp.where(gt, 5 * a, sh)
        inc = jnp.left_shift(jnp.ones_like(sh), sh)
        acc = inc if acc is None else acc + inc
    acc16 = acc[0:16]
    for k in range(1, PIX // 16):
        acc16 = acc16 + acc[k * 16:(k + 1) * 16]
    hs = []
    for a in range(N_ANGLE):
        f = jnp.bitwise_and(jnp.right_shift(acc16, 5 * a), 31)
        hs.append(jnp.sum(f.astype(jnp.float32), axis=0))
    h = jnp.stack(hs, axis=0).reshape(1, 1, N_ANGLE, x_ref.shape[-1])
    h_ref[...] = h
    v_ref[...] = jnp.sqrt(h * (1.0 / 6912.0))


def _tc_stats(xt):
    b, nf, npart, npix, n = xt.shape
    return pl.pallas_call(
        _tc_body,
        grid=(b, npart),
        in_specs=[
            pl.BlockSpec((1, nf, 1, npix, n), lambda i, q: (i, 0, q, 0, 0)),
        ],
        out_specs=[
            pl.BlockSpec((1, 1, N_ANGLE, n), lambda i, q: (i, q, 0, 0)),
            pl.BlockSpec((1, 1, N_ANGLE, n), lambda i, q: (i, q, 0, 0)),
        ],
        out_shape=[
            jax.ShapeDtypeStruct((b, npart, N_ANGLE, n), jnp.float32),
            jax.ShapeDtypeStruct((b, npart, N_ANGLE, n), jnp.float32),
        ],
    )(xt)


def _full(val):
    return jnp.full((GROUP,), val, jnp.int32)


def _sc_group(h_v, v_v, o_v):
    lanes = lax.iota(jnp.int32, GROUP)
    rows = [[h_v[q, a] for a in range(N_ANGLE)] for q in range(N_PARTS)]
    totals = []
    for a in range(N_ANGLE):
        t = rows[0][a]
        for q in range(1, N_PARTS):
            t = t + rows[q][a]
        totals.append(t)
    tmax = totals[0]
    for a in range(1, N_ANGLE):
        tmax = jnp.maximum(tmax, totals[a])
    pd = _full(0)
    for a in range(N_ANGLE - 1, -1, -1):
        pd = jnp.where(totals[a] == tmax, _full(a), pd)
    dis = []
    for ap in range(N_ANG_PARTS):
        d = None
        for rp in range(N_RAD_PARTS):
            g = plsc.load_gather(h_v, [_full(ap * 3 + rp), pd, lanes])
            d = g if d is None else d + g
        dis.append(d)
    dmax = dis[0]
    for ap in range(1, N_ANG_PARTS):
        dmax = jnp.maximum(dmax, dis[ap])
    pap = _full(0)
    for ap in range(N_ANG_PARTS - 1, -1, -1):
        pap = jnp.where(dis[ap] == dmax, _full(ap), pap)
    for j in range(N_ANG_PARTS):
        sap = jnp.remainder(pap + _full(j), _full(N_ANG_PARTS))
        for k in range(N_RAD_PARTS):
            qsrc = sap * 3 + _full(k)
            for l in range(N_ANGLE):
                sa = jnp.remainder(pd + _full(l), _full(N_ANGLE))
                val = plsc.load_gather(v_v, [qsrc, sa, lanes])
                plsc.store_scatter(
                    o_v, [lanes, _full(j * 18 + k * 6 + l)], val
                )


@functools.lru_cache(maxsize=2)
def _make_sc_kernel(b, n):
    groups_per_b = -(-n // GROUP)
    n_groups = b * groups_per_b
    steps = -(-n_groups // N_WORKERS)
    last_row = n - GROUP
    mesh = plsc.VectorSubcoreMesh(core_axis_name="c", subcore_axis_name="s")

    @functools.partial(
        pl.kernel,
        mesh=mesh,
        compiler_params=pltpu.CompilerParams(
            use_tc_tiling_on_sc=False, needs_layout_passes=False
        ),
        out_type=jax.ShapeDtypeStruct((b * n, OUT_W), jnp.float32),
        scratch_types=[
            pltpu.VMEM((N_PARTS, N_ANGLE, GROUP), jnp.float32),
            pltpu.VMEM((N_PARTS, N_ANGLE, GROUP), jnp.float32),
            pltpu.VMEM((GROUP, OUT_W), jnp.float32),
        ],
    )
    def sc_kernel(h_hbm, v_hbm, out_hbm, h_v, v_v, o_v):
        wid = lax.axis_index("s") * 2 + lax.axis_index("c")
        for gi in range(steps):
            g = gi * N_WORKERS + wid

            @pl.when(g < n_groups)
            def _():
                bb = g // groups_per_b
                gg = g % groups_per_b
                n0 = jnp.minimum(gg * GROUP, last_row)
                pltpu.sync_copy(h_hbm.at[bb, :, :, pl.ds(n0, GROUP)], h_v)
                pltpu.sync_copy(v_hbm.at[bb, :, :, pl.ds(n0, GROUP)], v_v)
                _sc_group(h_v, v_v, o_v)
                pltpu.sync_copy(o_v, out_hbm.at[pl.ds(bb * n + n0, GROUP), :])

    return sc_kernel


def kernel(patches):
    b, n = patches.shape[0], patches.shape[1]
    xt = jnp.transpose(patches, (0, 2, 3, 4, 1))
    h, v = _tc_stats(xt)
    out = _make_sc_kernel(b, n)(h, v)
    return out[:, :DESC].reshape(b, n, DESC)


# speedup vs baseline: 1.0447x; 1.0447x over previous
import functools

import jax
import jax.numpy as jnp
from jax import lax
from jax.experimental import pallas as pl
from jax.experimental.pallas import tpu as pltpu
from jax.experimental.pallas import tpu_sc as plsc

N_ANGLE = 6
N_SIGMA = 4
N_ANG_PARTS = 6
N_RAD_PARTS = 3
N_PARTS = N_ANG_PARTS * N_RAD_PARTS
PIX = 4 * 24
DESC = N_ANG_PARTS * N_RAD_PARTS * N_ANGLE

OUT_W = 128
GROUP = 16
N_WORKERS = 32


def _tc_body(x_ref, hv_ref):
    x = x_ref[...].reshape(N_ANGLE * N_SIGMA, PIX, x_ref.shape[-1])
    acc = None
    for s in range(N_SIGMA):
        m = x[s]
        sh = jnp.zeros(m.shape, jnp.int32)
        for a in range(1, N_ANGLE):
            xa = x[a * N_SIGMA + s]
            gt = xa > m
            m = jnp.maximum(m, xa)
            sh = jnp.where(gt, 5 * a, sh)
        inc = jnp.left_shift(jnp.ones_like(sh), sh)
        acc = inc if acc is None else acc + inc
    acc16 = acc[0:16]
    for k in range(1, PIX // 16):
        acc16 = acc16 + acc[k * 16:(k + 1) * 16]
    hs = []
    for a in range(N_ANGLE):
        f = jnp.bitwise_and(jnp.right_shift(acc16, 5 * a), 31)
        hs.append(jnp.sum(f.astype(jnp.float32), axis=0))
    h = jnp.stack(hs, axis=0)
    v = jnp.sqrt(h * (1.0 / 6912.0))
    hv_ref[...] = jnp.stack([h, v], axis=0).reshape(
        1, 2, 1, N_ANGLE, x_ref.shape[-1]
    )


def _tc_stats(xt):
    b, nf, npart, npix, n = xt.shape
    return pl.pallas_call(
        _tc_body,
        grid=(b, npart),
        in_specs=[
            pl.BlockSpec((1, nf, 1, npix, n), lambda i, q: (i, 0, q, 0, 0)),
        ],
        out_specs=pl.BlockSpec((1, 2, 1, N_ANGLE, n), lambda i, q: (i, 0, q, 0, 0)),
        out_shape=jax.ShapeDtypeStruct((b, 2, npart, N_ANGLE, n), jnp.float32),
    )(xt)


def _full(val):
    return jnp.full((GROUP,), val, jnp.int32)


def _sc_group(hv_v, o_v):
    lanes = lax.iota(jnp.int32, GROUP)
    rows = [[hv_v[0, q, a] for a in range(N_ANGLE)] for q in range(N_PARTS)]
    totals = []
    for a in range(N_ANGLE):
        t = rows[0][a]
        for q in range(1, N_PARTS):
            t = t + rows[q][a]
        totals.append(t)
    tmax = totals[0]
    for a in range(1, N_ANGLE):
        tmax = jnp.maximum(tmax, totals[a])
    pd = _full(0)
    for a in range(N_ANGLE - 1, -1, -1):
        pd = jnp.where(totals[a] == tmax, _full(a), pd)
    zero = _full(0)
    one = _full(1)
    dis = []
    for ap in range(N_ANG_PARTS):
        d = None
        for rp in range(N_RAD_PARTS):
            g = plsc.load_gather(hv_v, [zero, _full(ap * 3 + rp), pd, lanes])
            d = g if d is None else d + g
        dis.append(d)
    dmax = dis[0]
    for ap in range(1, N_ANG_PARTS):
        dmax = jnp.maximum(dmax, dis[ap])
    pap = _full(0)
    for ap in range(N_ANG_PARTS - 1, -1, -1):
        pap = jnp.where(dis[ap] == dmax, _full(ap), pap)
    for j in range(N_ANG_PARTS):
        sap = jnp.remainder(pap + _full(j), _full(N_ANG_PARTS))
        for k in range(N_RAD_PARTS):
            qsrc = sap * 3 + _full(k)
            for l in range(N_ANGLE):
                sa = jnp.remainder(pd + _full(l), _full(N_ANGLE))
                val = plsc.load_gather(hv_v, [one, qsrc, sa, lanes])
                plsc.store_scatter(
                    o_v, [lanes, _full(j * 18 + k * 6 + l)], val
                )


@functools.lru_cache(maxsize=2)
def _make_sc_kernel(b, n):
    groups_per_b = -(-n // GROUP)
    n_groups = b * groups_per_b
    steps = -(-n_groups // N_WORKERS)
    last_row = n - GROUP
    mesh = plsc.VectorSubcoreMesh(core_axis_name="c", subcore_axis_name="s")

    @functools.partial(
        pl.kernel,
        mesh=mesh,
        compiler_params=pltpu.CompilerParams(
            use_tc_tiling_on_sc=False, needs_layout_passes=False
        ),
        out_type=jax.ShapeDtypeStruct((b * n, OUT_W), jnp.float32),
        scratch_types=[
            pltpu.VMEM((2, N_PARTS, N_ANGLE, GROUP), jnp.float32),
            pltpu.VMEM((2, N_PARTS, N_ANGLE, GROUP), jnp.float32),
            pltpu.VMEM((GROUP, OUT_W), jnp.float32),
            pltpu.VMEM((GROUP, OUT_W), jnp.float32),
            pltpu.SemaphoreType.DMA,
            pltpu.SemaphoreType.DMA,
            pltpu.SemaphoreType.DMA,
            pltpu.SemaphoreType.DMA,
        ],
    )
    def sc_kernel(hv_hbm, out_hbm, hv_a, hv_b, o_a, o_b, si_a, si_b, so_a, so_b):
        wid = lax.axis_index("s") * 2 + lax.axis_index("c")
        in_bufs = [hv_a, hv_b]
        out_bufs = [o_a, o_b]
        in_sems = [si_a, si_b]
        out_sems = [so_a, so_b]

        def grp(gi):
            g = gi * N_WORKERS + wid
            bb = g // groups_per_b
            gg = g % groups_per_b
            n0 = jnp.minimum(gg * GROUP, last_row)
            return g < n_groups, bb, n0

        def in_copy(gi):
            ok, bb, n0 = grp(gi)
            slot = gi % 2
            return ok, pltpu.make_async_copy(
                hv_hbm.at[bb, :, :, :, pl.ds(n0, GROUP)], in_bufs[slot],
                in_sems[slot],
            )

        def out_copy(gi):
            ok, bb, n0 = grp(gi)
            slot = gi % 2
            return ok, pltpu.make_async_copy(
                out_bufs[slot], out_hbm.at[pl.ds(bb * n + n0, GROUP), :],
                out_sems[slot],
            )

        def start(ok_copy):
            ok, cp = ok_copy

            @pl.when(ok)
            def _():
                cp.start()

        def wait(ok_copy):
            ok, cp = ok_copy

            @pl.when(ok)
            def _():
                cp.wait()

        start(in_copy(0))
        for gi in range(steps):
            if gi + 1 < steps:
                start(in_copy(gi + 1))
            wait(in_copy(gi))
            if gi >= 2:
                wait(out_copy(gi - 2))
            ok, _, _ = grp(gi)

            @pl.when(ok)
            def _():
                _sc_group(in_bufs[gi % 2], out_bufs[gi % 2])

            start(out_copy(gi))
        for gi in range(max(0, steps - 2), steps):
            wait(out_copy(gi))

    return sc_kernel


def kernel(patches):
    b, n = patches.shape[0], patches.shape[1]
    xt = jnp.transpose(patches, (0, 2, 3, 4, 1))
    hv = _tc_stats(xt)
    out = _make_sc_kernel(b, n)(hv)
    return out[:, :DESC].reshape(b, n, DESC)
